# initial kernel scaffold (unmeasured)
import jax
import jax.numpy as jnp
from jax import lax
from jax.experimental import pallas as pl
from jax.experimental.pallas import tpu as pltpu

N_DEV = 4
T = 512
T_PER = 128
D = 512
F = 1024
E_PER = 2


def kernel(x, router, W1, W2):
    def body(x_ref, r_ref, w1_ref, w2_ref, out_ref,
             xg_ref, rg_ref, part_ref, cx_ref, cr_ref, cs_ref,
             x_send, x_recv, r_send, r_recv, s_send, s_recv):
        my = lax.axis_index("i")
        right = lax.rem(my + 1, N_DEV)
        left = lax.rem(my + N_DEV - 1, N_DEV)

        barrier_sem = pltpu.get_barrier_semaphore()
        for nbr in (left, right):
            pl.semaphore_signal(
                barrier_sem, inc=1,
                device_id=(nbr,), device_id_type=pl.DeviceIdType.MESH,
            )
        pl.semaphore_wait(barrier_sem, 2)

        xg_ref[pl.ds(my * T_PER, T_PER), :] = x_ref[...]
        rg_ref[pl.ds(my, 1)] = jnp.expand_dims(r_ref[...], 0)
        cx_ref[N_DEV - 1] = x_ref[...]
        cr_ref[N_DEV - 1] = r_ref[...]

        for h in range(N_DEV - 1):
            src = N_DEV - 1 if h == 0 else h - 1
            rdma_x = pltpu.make_async_remote_copy(
                src_ref=cx_ref.at[src], dst_ref=cx_ref.at[h],
                send_sem=x_send.at[h], recv_sem=x_recv.at[h],
                device_id=(right,), device_id_type=pl.DeviceIdType.MESH,
            )
            rdma_r = pltpu.make_async_remote_copy(
                src_ref=cr_ref.at[src], dst_ref=cr_ref.at[h],
                send_sem=r_send.at[h], recv_sem=r_recv.at[h],
                device_id=(right,), device_id_type=pl.DeviceIdType.MESH,
            )
            rdma_x.start()
            rdma_r.start()
            rdma_x.wait()
            rdma_r.wait()
            origin = lax.rem(my + N_DEV - 1 - h, N_DEV)
            xg_ref[pl.ds(origin * T_PER, T_PER), :] = cx_ref[h]
            rg_ref[pl.ds(origin, 1)] = cr_ref[pl.ds(h, 1)]

        xg = xg_ref[...]
        gates = jnp.concatenate(
            [jnp.dot(xg, rg_ref[j], preferred_element_type=jnp.float32)
             for j in range(N_DEV)],
            axis=1,
        )
        e_tot = N_DEV * E_PER
        iota = lax.broadcasted_iota(jnp.int32, (T, e_tot), 1)
        v1 = jnp.max(gates, axis=1, keepdims=True)
        i1 = jnp.min(jnp.where(gates == v1, iota, e_tot), axis=1,
                     keepdims=True)
        g2 = jnp.where(iota == i1, -1e30, gates)
        v2 = jnp.max(g2, axis=1, keepdims=True)
        i2 = jnp.min(jnp.where(g2 == v2, iota, e_tot), axis=1,
                     keepdims=True)
        b = jnp.exp(v2 - v1)
        w1p = 1.0 / (1.0 + b)
        w2p = b / (1.0 + b)

        part = jnp.zeros((T, D), jnp.float32)
        for j in range(E_PER):
            e = E_PER * my + j
            w_e = (jnp.where(i1 == e, w1p, 0.0)
                   + jnp.where(i2 == e, w2p, 0.0))
            h_act = jnp.maximum(
                jnp.dot(xg, w1_ref[j], preferred_element_type=jnp.float32),
                0.0)
            part = part + jnp.dot(
                h_act, w2_ref[j], preferred_element_type=jnp.float32) * w_e
        part_ref[...] = part

        cs_ref[N_DEV - 1] = part_ref[
            pl.ds(lax.rem(my + N_DEV - 1, N_DEV) * T_PER, T_PER), :]
        for s in range(N_DEV - 1):
            src = N_DEV - 1 if s == 0 else s - 1
            rdma = pltpu.make_async_remote_copy(
                src_ref=cs_ref.at[src], dst_ref=cs_ref.at[s],
                send_sem=s_send.at[s], recv_sem=s_recv.at[s],
                device_id=(right,), device_id_type=pl.DeviceIdType.MESH,
            )
            rdma.start()
            rdma.wait()
            c_r = lax.rem(my + N_DEV - 2 - s, N_DEV)
            if s < N_DEV - 2:
                cs_ref[s] = cs_ref[s] + part_ref[pl.ds(c_r * T_PER, T_PER), :]
            else:
                out_ref[...] = cs_ref[s] + part_ref[
                    pl.ds(c_r * T_PER, T_PER), :]

    return pl.pallas_call(
        body,
        out_shape=jax.ShapeDtypeStruct((T_PER, D), jnp.float32),
        in_specs=[pl.BlockSpec(memory_space=pltpu.VMEM)] * 4,
        out_specs=pl.BlockSpec(memory_space=pltpu.VMEM),
        scratch_shapes=[
            pltpu.VMEM((T, D), jnp.float32),
            pltpu.VMEM((N_DEV, D, E_PER), jnp.float32),
            pltpu.VMEM((T, D), jnp.float32),
            pltpu.VMEM((N_DEV, T_PER, D), jnp.float32),
            pltpu.VMEM((N_DEV, D, E_PER), jnp.float32),
            pltpu.VMEM((N_DEV, T_PER, D), jnp.float32),
            pltpu.SemaphoreType.DMA((N_DEV - 1,)),
            pltpu.SemaphoreType.DMA((N_DEV - 1,)),
            pltpu.SemaphoreType.DMA((N_DEV - 1,)),
            pltpu.SemaphoreType.DMA((N_DEV - 1,)),
            pltpu.SemaphoreType.DMA((N_DEV - 1,)),
            pltpu.SemaphoreType.DMA((N_DEV - 1,)),
        ],
        compiler_params=pltpu.CompilerParams(collective_id=0),
    )(x, router, W1, W2)


# baseline (device time: 63685 ns/iter reference)
import jax
import jax.numpy as jnp
from jax import lax
from jax.experimental import pallas as pl
from jax.experimental.pallas import tpu as pltpu

N_DEV = 4
T = 512
T_PER = 128
D = 512
F = 1024
E_PER = 2


def kernel(x, router, W1, W2):
    def body(x_ref, r_ref, w1_ref, w2_ref, out_ref,
             xg_ref, rg_ref, part_ref, cx_ref, cr_ref, cs_ref,
             x_send, x_recv, r_send, r_recv, s_send, s_recv):
        my = lax.axis_index("i")
        right = lax.rem(my + 1, N_DEV)
        left = lax.rem(my + N_DEV - 1, N_DEV)

        barrier_sem = pltpu.get_barrier_semaphore()
        for nbr in (left, right):
            pl.semaphore_signal(
                barrier_sem, inc=1,
                device_id=(nbr,), device_id_type=pl.DeviceIdType.MESH,
            )
        pl.semaphore_wait(barrier_sem, 2)

        xg_ref[pl.ds(my * T_PER, T_PER), :] = x_ref[...]
        rg_ref[pl.ds(my, 1)] = jnp.expand_dims(r_ref[...], 0)
        cx_ref[N_DEV - 1] = x_ref[...]
        cr_ref[N_DEV - 1] = r_ref[...]

        for h in range(N_DEV - 1):
            src = N_DEV - 1 if h == 0 else h - 1
            rdma_x = pltpu.make_async_remote_copy(
                src_ref=cx_ref.at[src], dst_ref=cx_ref.at[h],
                send_sem=x_send.at[h], recv_sem=x_recv.at[h],
                device_id=(right,), device_id_type=pl.DeviceIdType.MESH,
            )
            rdma_r = pltpu.make_async_remote_copy(
                src_ref=cr_ref.at[src], dst_ref=cr_ref.at[h],
                send_sem=r_send.at[h], recv_sem=r_recv.at[h],
                device_id=(right,), device_id_type=pl.DeviceIdType.MESH,
            )
            rdma_x.start()
            rdma_r.start()
            rdma_x.wait()
            rdma_r.wait()
            origin = lax.rem(my + N_DEV - 1 - h, N_DEV)
            xg_ref[pl.ds(origin * T_PER, T_PER), :] = cx_ref[h]
            rg_ref[pl.ds(origin, 1)] = cr_ref[pl.ds(h, 1)]

        xg = xg_ref[...]
        gates = jnp.concatenate(
            [jnp.dot(xg, rg_ref[j], preferred_element_type=jnp.float32,
                     precision=lax.Precision.HIGHEST)
             for j in range(N_DEV)],
            axis=1,
        )
        e_tot = N_DEV * E_PER
        iota = lax.broadcasted_iota(jnp.int32, (T, e_tot), 1)
        v1 = jnp.max(gates, axis=1, keepdims=True)
        i1 = jnp.min(jnp.where(gates == v1, iota, e_tot), axis=1,
                     keepdims=True)
        g2 = jnp.where(iota == i1, -1e30, gates)
        v2 = jnp.max(g2, axis=1, keepdims=True)
        i2 = jnp.min(jnp.where(g2 == v2, iota, e_tot), axis=1,
                     keepdims=True)
        b = jnp.exp(v2 - v1)
        w1p = 1.0 / (1.0 + b)
        w2p = b / (1.0 + b)

        part = jnp.zeros((T, D), jnp.float32)
        for j in range(E_PER):
            e = E_PER * my + j
            w_e = (jnp.where(i1 == e, w1p, 0.0)
                   + jnp.where(i2 == e, w2p, 0.0))
            h_act = jnp.maximum(
                jnp.dot(xg, w1_ref[j], preferred_element_type=jnp.float32,
                        precision=lax.Precision.HIGHEST),
                0.0)
            part = part + jnp.dot(
                h_act, w2_ref[j], preferred_element_type=jnp.float32,
                precision=lax.Precision.HIGHEST) * w_e
        part_ref[...] = part

        cs_ref[N_DEV - 1] = part_ref[
            pl.ds(lax.rem(my + N_DEV - 1, N_DEV) * T_PER, T_PER), :]
        for s in range(N_DEV - 1):
            src = N_DEV - 1 if s == 0 else s - 1
            rdma = pltpu.make_async_remote_copy(
                src_ref=cs_ref.at[src], dst_ref=cs_ref.at[s],
                send_sem=s_send.at[s], recv_sem=s_recv.at[s],
                device_id=(right,), device_id_type=pl.DeviceIdType.MESH,
            )
            rdma.start()
            rdma.wait()
            c_r = lax.rem(my + N_DEV - 2 - s, N_DEV)
            if s < N_DEV - 2:
                cs_ref[s] = cs_ref[s] + part_ref[pl.ds(c_r * T_PER, T_PER), :]
            else:
                out_ref[...] = cs_ref[s] + part_ref[
                    pl.ds(c_r * T_PER, T_PER), :]

    return pl.pallas_call(
        body,
        out_shape=jax.ShapeDtypeStruct((T_PER, D), jnp.float32),
        in_specs=[pl.BlockSpec(memory_space=pltpu.VMEM)] * 4,
        out_specs=pl.BlockSpec(memory_space=pltpu.VMEM),
        scratch_shapes=[
            pltpu.VMEM((T, D), jnp.float32),
            pltpu.VMEM((N_DEV, D, E_PER), jnp.float32),
            pltpu.VMEM((T, D), jnp.float32),
            pltpu.VMEM((N_DEV, T_PER, D), jnp.float32),
            pltpu.VMEM((N_DEV, D, E_PER), jnp.float32),
            pltpu.VMEM((N_DEV, T_PER, D), jnp.float32),
            pltpu.SemaphoreType.DMA((N_DEV - 1,)),
            pltpu.SemaphoreType.DMA((N_DEV - 1,)),
            pltpu.SemaphoreType.DMA((N_DEV - 1,)),
            pltpu.SemaphoreType.DMA((N_DEV - 1,)),
            pltpu.SemaphoreType.DMA((N_DEV - 1,)),
            pltpu.SemaphoreType.DMA((N_DEV - 1,)),
        ],
        compiler_params=pltpu.CompilerParams(collective_id=0),
    )(x, router, W1, W2)
